# baseline (speedup 1.0000x reference)
import functools

import jax
import jax.numpy as jnp
from jax import lax
from jax.experimental import pallas as pl
from jax.experimental.pallas import tpu as pltpu
from jax.experimental.pallas import tpu_sc as plsc

GRID_LOWER = (0.0, 0.0, 0.0)
GRID_STEPS = (0.015625, 0.015625, 0.015625)

NC, NS, L = 2, 16, 16
NW = NC * NS
CHUNK = 128
GROUPS = CHUNK // L


def _bcast_gather(v, idx):
    dn = lax.GatherDimensionNumbers(
        offset_dims=(), collapsed_slice_dims=(0,), start_index_map=(0,))
    return lax.gather(v, idx[:, None], dn, (1,),
                      mode=lax.GatherScatterMode.PROMISE_IN_BOUNDS)


def _bcast_lane(v, lane):
    return _bcast_gather(v, jnp.full((L,), lane, jnp.int32))


def _axis_coords(cv, hi):
    t = (cv + 64.0).astype(jnp.int32)
    i0 = t - 64
    f = cv - i0.astype(jnp.float32)
    i0c = jnp.clip(i0, 0, hi)
    i1c = jnp.clip(i0 + 1, 0, hi)
    return i0c, i1c, f


def _make_sc_call(XYZ, NPB, npt, x_dim, y_dim, z_dim, C, N):
    nchunks = npt // CHUNK
    mesh = plsc.VectorSubcoreMesh(
        core_axis_name="c", subcore_axis_name="s",
        num_cores=NC, num_subcores=NS)
    B = (NW * npt) // NPB
    inv_step = 1.0 / GRID_STEPS[0]
    PART = N % CHUNK

    @functools.partial(
        pl.kernel,
        out_type=jax.ShapeDtypeStruct((B * N, C), jnp.float32),
        mesh=mesh,
        scratch_types=[
            pltpu.VMEM((npt * 3,), jnp.float32),
            pltpu.VMEM((8, CHUNK), jnp.int32),
            pltpu.VMEM((8, CHUNK), jnp.int32),
            pltpu.VMEM((8, CHUNK, C), jnp.float32),
            pltpu.VMEM((8, CHUNK, C), jnp.float32),
            pltpu.VMEM((3, CHUNK), jnp.float32),
            pltpu.VMEM((3, CHUNK), jnp.float32),
            pltpu.VMEM((CHUNK, C), jnp.float32),
            pltpu.VMEM((CHUNK, C), jnp.float32),
            pltpu.SemaphoreType.DMA,
            pltpu.SemaphoreType.DMA,
        ],
        compiler_params=pltpu.CompilerParams(use_tc_tiling_on_sc=False),
    )
    def sc_call(gflat, locs_flat, out,
                cv, idxA, idxB, rowsA, rowsB, frA, frB,
                outA, outB, semA, semB):
        cid = lax.axis_index("c")
        sid = lax.axis_index("s")
        wid = sid * NC + cid
        base = wid * npt
        batch = base // NPB
        boff = batch * XYZ
        inb0 = base - batch * NPB
        orow0 = batch * N

        pltpu.sync_copy(locs_flat.at[pl.ds(base * 3, npt * 3)], cv)

        lanes = lax.iota(jnp.int32, L)

        def deint(q, a):
            pos = 3 * lanes + a
            idx = pos & 15
            j = pos >> 4
            g0 = _bcast_gather(cv[pl.ds(q, L)], idx)
            g1 = _bcast_gather(cv[pl.ds(q + L, L)], idx)
            g2 = _bcast_gather(cv[pl.ds(q + 2 * L, L)], idx)
            return jnp.where(j == 0, g0, jnp.where(j == 1, g1, g2))

        def stage(c, idx_r, fr_r):
            off = c * CHUNK
            for g in range(GROUPS):
                s = off + g * L
                q = s * 3
                cx = deint(q, 0) * inv_step - 0.5
                cy = deint(q, 1) * inv_step - 0.5
                cz = deint(q, 2) * inv_step - 0.5
                x0, x1, fx = _axis_coords(cx, x_dim - 1)
                y0, y1, fy = _axis_coords(cy, y_dim - 1)
                z0, z1, fz = _axis_coords(cz, z_dim - 1)
                xb = (x0 * (y_dim * z_dim) + boff, x1 * (y_dim * z_dim) + boff)
                yb = (y0 * z_dim, y1 * z_dim)
                zb = (z0, z1)
                fr_r[0, pl.ds(g * L, L)] = fx
                fr_r[1, pl.ds(g * L, L)] = fy
                fr_r[2, pl.ds(g * L, L)] = fz
                for dx in (0, 1):
                    xyb = (xb[dx] + yb[0], xb[dx] + yb[1])
                    for dy in (0, 1):
                        for dz in (0, 1):
                            k = dx * 4 + dy * 2 + dz
                            idx_r[k, pl.ds(g * L, L)] = xyb[dy] + zb[dz]

        def fire(idx_r, rows_r, sem):
            for k in range(8):
                pltpu.async_copy(gflat.at[idx_r.at[k]], rows_r.at[k], sem)

        def drain(idx_r, rows_r, sem):
            for k in range(8):
                pltpu.make_async_copy(gflat.at[idx_r.at[k]], rows_r.at[k],
                                      sem).wait()

        def accum(c, rows_r, fr_r, ob):
            def pbody(p, _):
                gb = (p // L) * L
                lane = p - gb
                bx = _bcast_lane(fr_r[0, pl.ds(gb, L)], lane)
                by = _bcast_lane(fr_r[1, pl.ds(gb, L)], lane)
                bz = _bcast_lane(fr_r[2, pl.ds(gb, L)], lane)
                wx = (1.0 - bx, bx)
                wyz = ((1.0 - by) * (1.0 - bz), (1.0 - by) * bz,
                       by * (1.0 - bz), by * bz)
                w0 = wx[0] * wyz[0]
                a0 = rows_r[0, p, pl.ds(0, L)] * w0
                a1 = rows_r[0, p, pl.ds(L, L)] * w0
                for k in range(1, 8):
                    wk = wx[k // 4] * wyz[k % 4]
                    a0 = a0 + rows_r[k, p, pl.ds(0, L)] * wk
                    a1 = a1 + rows_r[k, p, pl.ds(L, L)] * wk
                ob[p, pl.ds(0, L)] = a0
                ob[p, pl.ds(L, L)] = a1
                return _

            lax.fori_loop(0, CHUNK, pbody, 0, unroll=2)
            inb = inb0 + c * CHUNK
            orow = orow0 + inb

            @pl.when(inb + CHUNK <= N)
            def _full():
                pltpu.sync_copy(ob, out.at[pl.ds(orow, CHUNK)])

            if PART:
                @pl.when(jnp.logical_and(inb < N, inb + CHUNK > N))
                def _partial():
                    pltpu.sync_copy(ob.at[pl.ds(0, PART)],
                                    out.at[pl.ds(orow, PART)])

        stage(0, idxA, frA)
        fire(idxA, rowsA, semA)

        def pair(i, _):
            c0 = i * 2
            stage(c0 + 1, idxB, frB)
            fire(idxB, rowsB, semB)
            drain(idxA, rowsA, semA)
            accum(c0, rowsA, frA, outA)

            @pl.when(i + 1 < nchunks // 2)
            def _fire_next():
                stage(c0 + 2, idxA, frA)
                fire(idxA, rowsA, semA)

            drain(idxB, rowsB, semB)
            accum(c0 + 1, rowsB, frB, outB)
            return _

        lax.fori_loop(0, nchunks // 2, pair, 0)

    return sc_call


def kernel(grid, locs):
    B, X, Y, Z, C = grid.shape
    N = locs.shape[1]
    NPB = -(-N // (NW * CHUNK * 2)) * (NW * CHUNK * 2)
    npt = (B * NPB) // NW

    gflat = grid.reshape(B * X * Y * Z, C)
    pad = NPB - N
    locs_p = jnp.pad(locs.astype(jnp.float32), ((0, 0), (0, pad), (0, 0)))
    locs_flat = locs_p.reshape(B * NPB * 3)

    sc_call = _make_sc_call(X * Y * Z, NPB, npt, X, Y, Z, C, N)
    out_p = sc_call(gflat, locs_flat)
    return out_p.reshape(B, N, C)


# speedup vs baseline: 2.1190x; 2.1190x over previous
import functools

import jax
import jax.numpy as jnp
from jax import lax
from jax.experimental import pallas as pl
from jax.experimental.pallas import tpu as pltpu
from jax.experimental.pallas import tpu_sc as plsc

GRID_LOWER = (0.0, 0.0, 0.0)
GRID_STEPS = (0.015625, 0.015625, 0.015625)

NC, NS, L = 2, 16, 16
CHUNK = 128
GROUPS = CHUNK // L
CORE0_FRAC = 0.81


def _bcast_lane(v, lane):
    idx = jnp.full((L,), lane, jnp.int32)
    dn = lax.GatherDimensionNumbers(
        offset_dims=(), collapsed_slice_dims=(0,), start_index_map=(0,))
    return lax.gather(v, idx[:, None], dn, (1,),
                      mode=lax.GatherScatterMode.PROMISE_IN_BOUNDS)


def _axis_coords(cv, hi):
    t = (cv + 64.0).astype(jnp.int32)
    i0 = t - 64
    f = cv - i0.astype(jnp.float32)
    i0c = jnp.clip(i0, 0, hi)
    i1c = jnp.clip(i0 + 1, 0, hi)
    return i0c, i1c, f


def _make_sc_call(B, N, x_dim, y_dim, z_dim, C):
    XYZ = x_dim * y_dim * z_dim
    slots = NS // B
    per_batch = N // slots
    s0 = int(round(CORE0_FRAC * per_batch / (2 * CHUNK))) * (2 * CHUNK)
    s0 = max(2 * CHUNK, min(s0, per_batch - 8))
    s1 = per_batch - s0
    nch0 = s0 // CHUNK
    nch1 = -(-s1 // CHUNK)
    if nch1 % 2:
        nch1 += 1
    part1 = s1 - (s1 // CHUNK) * CHUNK
    mesh = plsc.VectorSubcoreMesh(
        core_axis_name="c", subcore_axis_name="s",
        num_cores=NC, num_subcores=NS)
    inv_step = 1.0 / GRID_STEPS[0]

    @functools.partial(
        pl.kernel,
        out_type=jax.ShapeDtypeStruct((B * N, C), jnp.float32),
        mesh=mesh,
        scratch_types=[
            pltpu.VMEM((8, CHUNK), jnp.int32),
            pltpu.VMEM((8, CHUNK), jnp.int32),
            pltpu.VMEM((8, CHUNK, C), jnp.float32),
            pltpu.VMEM((8, CHUNK, C), jnp.float32),
            pltpu.VMEM((3, CHUNK), jnp.float32),
            pltpu.VMEM((3, CHUNK), jnp.float32),
            pltpu.VMEM((3, CHUNK), jnp.float32),
            pltpu.VMEM((3, CHUNK), jnp.float32),
            pltpu.VMEM((CHUNK, C), jnp.float32),
            pltpu.VMEM((CHUNK, C), jnp.float32),
            pltpu.SemaphoreType.DMA,
            pltpu.SemaphoreType.DMA,
        ],
        compiler_params=pltpu.CompilerParams(use_tc_tiling_on_sc=False),
    )
    def sc_call(gflat, xs, ys, zs, out,
                idxA, idxB, rowsA, rowsB, cvA, cvB, frA, frB,
                outA, outB, semA, semB):
        cid = lax.axis_index("c")
        sid = lax.axis_index("s")
        batch = sid // slots
        slot = sid - batch * slots
        is0 = cid == 0
        sb = jnp.where(is0, slot * s0, slots * s0 + slot * s1)
        slen = jnp.where(is0, s0, s1)
        npairs = jnp.where(is0, nch0 // 2, nch1 // 2)
        boff = batch * XYZ
        orow0 = batch * N + sb

        def coff(c):
            return jnp.minimum(c * CHUNK, slen - CHUNK)

        def fetch_coords(c, cv_r):
            o = sb + coff(c)
            pltpu.async_copy(xs.at[batch, pl.ds(o, CHUNK)],
                             cv_r.at[0], semA if cv_r is cvA else semB)
            pltpu.async_copy(ys.at[batch, pl.ds(o, CHUNK)],
                             cv_r.at[1], semA if cv_r is cvA else semB)
            pltpu.async_copy(zs.at[batch, pl.ds(o, CHUNK)],
                             cv_r.at[2], semA if cv_r is cvA else semB)

        def fetch_coords_sync(c, cv_r):
            o = sb + coff(c)
            pltpu.sync_copy(xs.at[batch, pl.ds(o, CHUNK)], cv_r.at[0])
            pltpu.sync_copy(ys.at[batch, pl.ds(o, CHUNK)], cv_r.at[1])
            pltpu.sync_copy(zs.at[batch, pl.ds(o, CHUNK)], cv_r.at[2])

        def stage(idx_r, cv_r, fr_r):
            for g in range(GROUPS):
                cx = cv_r[0, pl.ds(g * L, L)] * inv_step - 0.5
                cy = cv_r[1, pl.ds(g * L, L)] * inv_step - 0.5
                cz = cv_r[2, pl.ds(g * L, L)] * inv_step - 0.5
                x0, x1, fx = _axis_coords(cx, x_dim - 1)
                y0, y1, fy = _axis_coords(cy, y_dim - 1)
                z0, z1, fz = _axis_coords(cz, z_dim - 1)
                xb = (x0 * (y_dim * z_dim) + boff, x1 * (y_dim * z_dim) + boff)
                yb = (y0 * z_dim, y1 * z_dim)
                zb = (z0, z1)
                fr_r[0, pl.ds(g * L, L)] = fx
                fr_r[1, pl.ds(g * L, L)] = fy
                fr_r[2, pl.ds(g * L, L)] = fz
                for dx in (0, 1):
                    xyb = (xb[dx] + yb[0], xb[dx] + yb[1])
                    for dy in (0, 1):
                        for dz in (0, 1):
                            k = dx * 4 + dy * 2 + dz
                            idx_r[k, pl.ds(g * L, L)] = xyb[dy] + zb[dz]

        def fire(c, idx_r, rows_r, cv_r, sem):
            for k in range(8):
                pltpu.async_copy(gflat.at[idx_r.at[k]], rows_r.at[k], sem)
            fetch_coords(c + 2, cv_r)

        def drain(c, idx_r, rows_r, cv_r, sem):
            for k in range(8):
                pltpu.make_async_copy(gflat.at[idx_r.at[k]], rows_r.at[k],
                                      sem).wait()
            o = sb + coff(c + 2)
            pltpu.make_async_copy(xs.at[batch, pl.ds(o, CHUNK)],
                                  cv_r.at[0], sem).wait()
            pltpu.make_async_copy(ys.at[batch, pl.ds(o, CHUNK)],
                                  cv_r.at[1], sem).wait()
            pltpu.make_async_copy(zs.at[batch, pl.ds(o, CHUNK)],
                                  cv_r.at[2], sem).wait()

        def accum(c, rows_r, fr_r, ob):
            def pbody(p, carry):
                gb = (p // L) * L
                lane = p - gb
                bx = _bcast_lane(fr_r[0, pl.ds(gb, L)], lane)
                by = _bcast_lane(fr_r[1, pl.ds(gb, L)], lane)
                bz = _bcast_lane(fr_r[2, pl.ds(gb, L)], lane)
                wx = (1.0 - bx, bx)
                wyz = ((1.0 - by) * (1.0 - bz), (1.0 - by) * bz,
                       by * (1.0 - bz), by * bz)
                w0 = wx[0] * wyz[0]
                a0 = rows_r[0, p, pl.ds(0, L)] * w0
                a1 = rows_r[0, p, pl.ds(L, L)] * w0
                for k in range(1, 8):
                    wk = wx[k // 4] * wyz[k % 4]
                    a0 = a0 + rows_r[k, p, pl.ds(0, L)] * wk
                    a1 = a1 + rows_r[k, p, pl.ds(L, L)] * wk
                ob[p, pl.ds(0, L)] = a0
                ob[p, pl.ds(L, L)] = a1
                return carry

            lax.fori_loop(0, CHUNK, pbody, 0, unroll=2)
            full = c * CHUNK + CHUNK <= slen

            @pl.when(full)
            def _full():
                pltpu.sync_copy(ob, out.at[pl.ds(orow0 + c * CHUNK, CHUNK)])

            if part1:
                @pl.when(jnp.logical_not(full))
                def _partial():
                    pltpu.sync_copy(
                        ob.at[pl.ds(CHUNK - part1, part1)],
                        out.at[pl.ds(orow0 + slen - part1, part1)])

        def pair(i, carry):
            c0 = i * 2
            stage(idxB, cvB, frB)
            fire(c0 + 1, idxB, rowsB, cvB, semB)
            drain(c0, idxA, rowsA, cvA, semA)
            accum(c0, rowsA, frA, outA)

            @pl.when(i + 1 < npairs)
            def _next_even():
                stage(idxA, cvA, frA)
                fire(c0 + 2, idxA, rowsA, cvA, semA)

            drain(c0 + 1, idxB, rowsB, cvB, semB)
            accum(c0 + 1, rowsB, frB, outB)
            return carry

        fetch_coords_sync(0, cvA)
        fetch_coords_sync(1, cvB)
        stage(idxA, cvA, frA)
        fire(0, idxA, rowsA, cvA, semA)
        lax.fori_loop(0, npairs, pair, 0)

    return sc_call


def kernel(grid, locs):
    B, X, Y, Z, C = grid.shape
    N = locs.shape[1]
    gflat = grid.reshape(B * X * Y * Z, C)
    locs32 = locs.astype(jnp.float32)
    xs = locs32[:, :, 0]
    ys = locs32[:, :, 1]
    zs = locs32[:, :, 2]
    sc_call = _make_sc_call(B, N, X, Y, Z, C)
    out = sc_call(gflat, xs, ys, zs)
    return out.reshape(B, N, C)


# speedup vs baseline: 2.4896x; 1.1749x over previous
import functools

import jax
import jax.numpy as jnp
from jax import lax
from jax.experimental import pallas as pl
from jax.experimental.pallas import tpu as pltpu
from jax.experimental.pallas import tpu_sc as plsc

GRID_LOWER = (0.0, 0.0, 0.0)
GRID_STEPS = (0.015625, 0.015625, 0.015625)

NC, NS, L = 2, 16, 16
CHUNK = 128
GROUPS = CHUNK // L
CORE0_FRAC = 0.5


def _bcast_lane(v, lane):
    idx = jnp.full((L,), lane, jnp.int32)
    dn = lax.GatherDimensionNumbers(
        offset_dims=(), collapsed_slice_dims=(0,), start_index_map=(0,))
    return lax.gather(v, idx[:, None], dn, (1,),
                      mode=lax.GatherScatterMode.PROMISE_IN_BOUNDS)


def _axis_coords(cv, hi):
    t = (cv + 64.0).astype(jnp.int32)
    i0 = t - 64
    f = cv - i0.astype(jnp.float32)
    i0c = jnp.clip(i0, 0, hi)
    i1c = jnp.clip(i0 + 1, 0, hi)
    return i0c, i1c, f


def _make_sc_call(B, N, x_dim, y_dim, z_dim, C):
    XYZ = x_dim * y_dim * z_dim
    slots = NS // B
    per_batch = N // slots
    s0 = int(round(CORE0_FRAC * per_batch / (2 * CHUNK))) * (2 * CHUNK)
    s0 = max(2 * CHUNK, min(s0, per_batch - 8))
    s1 = per_batch - s0
    nch0 = s0 // CHUNK
    nch1 = -(-s1 // CHUNK)
    if nch1 % 2:
        nch1 += 1
    part1 = s1 - (s1 // CHUNK) * CHUNK
    mesh = plsc.VectorSubcoreMesh(
        core_axis_name="c", subcore_axis_name="s",
        num_cores=NC, num_subcores=NS)
    inv_step = 1.0 / GRID_STEPS[0]

    @functools.partial(
        pl.kernel,
        out_type=jax.ShapeDtypeStruct((B * N, C), jnp.float32),
        mesh=mesh,
        scratch_types=[
            pltpu.VMEM((8, CHUNK), jnp.int32),
            pltpu.VMEM((8, CHUNK), jnp.int32),
            pltpu.VMEM((8, CHUNK, C), jnp.float32),
            pltpu.VMEM((8, CHUNK, C), jnp.float32),
            pltpu.VMEM((3, CHUNK), jnp.float32),
            pltpu.VMEM((3, CHUNK), jnp.float32),
            pltpu.VMEM((3, CHUNK), jnp.float32),
            pltpu.VMEM((3, CHUNK), jnp.float32),
            pltpu.VMEM((CHUNK, C), jnp.float32),
            pltpu.VMEM((CHUNK, C), jnp.float32),
            pltpu.SemaphoreType.DMA,
            pltpu.SemaphoreType.DMA,
        ],
        compiler_params=pltpu.CompilerParams(use_tc_tiling_on_sc=False),
    )
    def sc_call(gflat, xs, ys, zs, out,
                idxA, idxB, rowsA, rowsB, cvA, cvB, frA, frB,
                outA, outB, semA, semB):
        cid = lax.axis_index("c")
        sid = lax.axis_index("s")
        batch = sid // slots
        slot = sid - batch * slots
        is0 = cid == 0
        sb = jnp.where(is0, slot * s0, slots * s0 + slot * s1)
        slen = jnp.where(is0, s0, s1)
        npairs = jnp.where(is0, nch0 // 2, nch1 // 2)
        boff = batch * XYZ
        orow0 = batch * N + sb

        def coff(c):
            return jnp.minimum(c * CHUNK, slen - CHUNK)

        def fetch_coords(c, cv_r):
            o = sb + coff(c)
            pltpu.async_copy(xs.at[batch, pl.ds(o, CHUNK)],
                             cv_r.at[0], semA if cv_r is cvA else semB)
            pltpu.async_copy(ys.at[batch, pl.ds(o, CHUNK)],
                             cv_r.at[1], semA if cv_r is cvA else semB)
            pltpu.async_copy(zs.at[batch, pl.ds(o, CHUNK)],
                             cv_r.at[2], semA if cv_r is cvA else semB)

        def fetch_coords_sync(c, cv_r):
            o = sb + coff(c)
            pltpu.sync_copy(xs.at[batch, pl.ds(o, CHUNK)], cv_r.at[0])
            pltpu.sync_copy(ys.at[batch, pl.ds(o, CHUNK)], cv_r.at[1])
            pltpu.sync_copy(zs.at[batch, pl.ds(o, CHUNK)], cv_r.at[2])

        def stage(idx_r, cv_r, fr_r):
            for g in range(GROUPS):
                cx = cv_r[0, pl.ds(g * L, L)] * inv_step - 0.5
                cy = cv_r[1, pl.ds(g * L, L)] * inv_step - 0.5
                cz = cv_r[2, pl.ds(g * L, L)] * inv_step - 0.5
                x0, x1, fx = _axis_coords(cx, x_dim - 1)
                y0, y1, fy = _axis_coords(cy, y_dim - 1)
                z0, z1, fz = _axis_coords(cz, z_dim - 1)
                xb = (x0 * (y_dim * z_dim) + boff, x1 * (y_dim * z_dim) + boff)
                yb = (y0 * z_dim, y1 * z_dim)
                zb = (z0, z1)
                fr_r[0, pl.ds(g * L, L)] = fx
                fr_r[1, pl.ds(g * L, L)] = fy
                fr_r[2, pl.ds(g * L, L)] = fz
                for dx in (0, 1):
                    xyb = (xb[dx] + yb[0], xb[dx] + yb[1])
                    for dy in (0, 1):
                        for dz in (0, 1):
                            k = dx * 4 + dy * 2 + dz
                            idx_r[k, pl.ds(g * L, L)] = xyb[dy] + zb[dz]

        def fire(c, idx_r, rows_r, cv_r, sem):
            for k in range(8):
                pltpu.async_copy(gflat.at[idx_r.at[k]], rows_r.at[k], sem)
            fetch_coords(c + 2, cv_r)

        def drain(c, idx_r, rows_r, cv_r, sem):
            for k in range(8):
                pltpu.make_async_copy(gflat.at[idx_r.at[k]], rows_r.at[k],
                                      sem).wait()
            o = sb + coff(c + 2)
            pltpu.make_async_copy(xs.at[batch, pl.ds(o, CHUNK)],
                                  cv_r.at[0], sem).wait()
            pltpu.make_async_copy(ys.at[batch, pl.ds(o, CHUNK)],
                                  cv_r.at[1], sem).wait()
            pltpu.make_async_copy(zs.at[batch, pl.ds(o, CHUNK)],
                                  cv_r.at[2], sem).wait()

        def accum(c, rows_r, fr_r, ob):
            def pbody(p, carry):
                gb = (p // L) * L
                lane = p - gb
                bx = _bcast_lane(fr_r[0, pl.ds(gb, L)], lane)
                by = _bcast_lane(fr_r[1, pl.ds(gb, L)], lane)
                bz = _bcast_lane(fr_r[2, pl.ds(gb, L)], lane)
                wx = (1.0 - bx, bx)
                wyz = ((1.0 - by) * (1.0 - bz), (1.0 - by) * bz,
                       by * (1.0 - bz), by * bz)
                w0 = wx[0] * wyz[0]
                a0 = rows_r[0, p, pl.ds(0, L)] * w0
                a1 = rows_r[0, p, pl.ds(L, L)] * w0
                for k in range(1, 8):
                    wk = wx[k // 4] * wyz[k % 4]
                    a0 = a0 + rows_r[k, p, pl.ds(0, L)] * wk
                    a1 = a1 + rows_r[k, p, pl.ds(L, L)] * wk
                ob[p, pl.ds(0, L)] = a0
                ob[p, pl.ds(L, L)] = a1
                return carry

            lax.fori_loop(0, CHUNK, pbody, 0, unroll=2)
            full = c * CHUNK + CHUNK <= slen

            @pl.when(full)
            def _full():
                pltpu.sync_copy(ob, out.at[pl.ds(orow0 + c * CHUNK, CHUNK)])

            if part1:
                @pl.when(jnp.logical_not(full))
                def _partial():
                    pltpu.sync_copy(
                        ob.at[pl.ds(CHUNK - part1, part1)],
                        out.at[pl.ds(orow0 + slen - part1, part1)])

        def pair(i, carry):
            c0 = i * 2
            stage(idxB, cvB, frB)
            fire(c0 + 1, idxB, rowsB, cvB, semB)
            drain(c0, idxA, rowsA, cvA, semA)
            accum(c0, rowsA, frA, outA)

            @pl.when(i + 1 < npairs)
            def _next_even():
                stage(idxA, cvA, frA)
                fire(c0 + 2, idxA, rowsA, cvA, semA)

            drain(c0 + 1, idxB, rowsB, cvB, semB)
            accum(c0 + 1, rowsB, frB, outB)
            return carry

        fetch_coords_sync(0, cvA)
        fetch_coords_sync(1, cvB)
        stage(idxA, cvA, frA)
        fire(0, idxA, rowsA, cvA, semA)
        lax.fori_loop(0, npairs, pair, 0)

    return sc_call


def kernel(grid, locs):
    B, X, Y, Z, C = grid.shape
    N = locs.shape[1]
    gflat = grid.reshape(B * X * Y * Z, C)
    locs32 = locs.astype(jnp.float32)
    xs = locs32[:, :, 0]
    ys = locs32[:, :, 1]
    zs = locs32[:, :, 2]
    sc_call = _make_sc_call(B, N, X, Y, Z, C)
    out = sc_call(gflat, xs, ys, zs)
    return out.reshape(B, N, C)


# speedup vs baseline: 2.9779x; 1.1961x over previous
import functools

import jax
import jax.numpy as jnp
from jax import lax
from jax.experimental import pallas as pl
from jax.experimental.pallas import tpu as pltpu
from jax.experimental.pallas import tpu_sc as plsc

GRID_LOWER = (0.0, 0.0, 0.0)
GRID_STEPS = (0.015625, 0.015625, 0.015625)

NC, NS, L = 2, 16, 16
CHUNK = 128
GROUPS = CHUNK // L
CORE0_FRAC = 0.5


def _bcast_lane(v, lane):
    idx = jnp.full((L,), lane, jnp.int32)
    dn = lax.GatherDimensionNumbers(
        offset_dims=(), collapsed_slice_dims=(0,), start_index_map=(0,))
    return lax.gather(v, idx[:, None], dn, (1,),
                      mode=lax.GatherScatterMode.PROMISE_IN_BOUNDS)


def _axis_coords(cv, hi):
    t = (cv + 64.0).astype(jnp.int32)
    i0 = t - 64
    f = cv - i0.astype(jnp.float32)
    i0c = jnp.clip(i0, 0, hi)
    i1c = jnp.clip(i0 + 1, 0, hi)
    return i0c, i1c, f


def _make_sc_call(B, N, x_dim, y_dim, z_dim, C):
    XYZ = x_dim * y_dim * z_dim
    slots = NS // B
    per_batch = N // slots
    s0 = int(round(CORE0_FRAC * per_batch / (2 * CHUNK))) * (2 * CHUNK)
    s0 = max(2 * CHUNK, min(s0, per_batch - 8))
    s1 = per_batch - s0
    nch0 = s0 // CHUNK
    nch1 = -(-s1 // CHUNK)
    if nch1 % 2:
        nch1 += 1
    part1 = s1 - (s1 // CHUNK) * CHUNK
    mesh = plsc.VectorSubcoreMesh(
        core_axis_name="c", subcore_axis_name="s",
        num_cores=NC, num_subcores=NS)
    inv_step = 1.0 / GRID_STEPS[0]

    @functools.partial(
        pl.kernel,
        out_type=jax.ShapeDtypeStruct((B, N, C), jnp.float32),
        mesh=mesh,
        scratch_types=[
            pltpu.VMEM((8, CHUNK), jnp.int32),
            pltpu.VMEM((8, CHUNK), jnp.int32),
            pltpu.VMEM((8, CHUNK, C), jnp.float32),
            pltpu.VMEM((8, CHUNK, C), jnp.float32),
            pltpu.VMEM((3, CHUNK), jnp.float32),
            pltpu.VMEM((3, CHUNK), jnp.float32),
            pltpu.VMEM((3, CHUNK), jnp.float32),
            pltpu.VMEM((3, CHUNK), jnp.float32),
            pltpu.VMEM((CHUNK, C), jnp.float32),
            pltpu.VMEM((CHUNK, C), jnp.float32),
            pltpu.SemaphoreType.DMA,
            pltpu.SemaphoreType.DMA,
        ],
        compiler_params=pltpu.CompilerParams(use_tc_tiling_on_sc=False),
    )
    def sc_call(gflat, xs, ys, zs, out,
                idxA, idxB, rowsA, rowsB, cvA, cvB, frA, frB,
                outA, outB, semA, semB):
        cid = lax.axis_index("c")
        sid = lax.axis_index("s")
        batch = sid // slots
        slot = sid - batch * slots
        is0 = cid == 0
        sb = jnp.where(is0, slot * s0, slots * s0 + slot * s1)
        slen = jnp.where(is0, s0, s1)
        npairs = jnp.where(is0, nch0 // 2, nch1 // 2)
        boff = batch * XYZ
        orow0 = sb

        def coff(c):
            return jnp.minimum(c * CHUNK, slen - CHUNK)

        def fetch_coords(c, cv_r):
            o = sb + coff(c)
            pltpu.async_copy(xs.at[batch, pl.ds(o, CHUNK)],
                             cv_r.at[0], semA if cv_r is cvA else semB)
            pltpu.async_copy(ys.at[batch, pl.ds(o, CHUNK)],
                             cv_r.at[1], semA if cv_r is cvA else semB)
            pltpu.async_copy(zs.at[batch, pl.ds(o, CHUNK)],
                             cv_r.at[2], semA if cv_r is cvA else semB)

        def fetch_coords_sync(c, cv_r):
            o = sb + coff(c)
            pltpu.sync_copy(xs.at[batch, pl.ds(o, CHUNK)], cv_r.at[0])
            pltpu.sync_copy(ys.at[batch, pl.ds(o, CHUNK)], cv_r.at[1])
            pltpu.sync_copy(zs.at[batch, pl.ds(o, CHUNK)], cv_r.at[2])

        def stage(idx_r, cv_r, fr_r):
            for g in range(GROUPS):
                cx = cv_r[0, pl.ds(g * L, L)] * inv_step - 0.5
                cy = cv_r[1, pl.ds(g * L, L)] * inv_step - 0.5
                cz = cv_r[2, pl.ds(g * L, L)] * inv_step - 0.5
                x0, x1, fx = _axis_coords(cx, x_dim - 1)
                y0, y1, fy = _axis_coords(cy, y_dim - 1)
                z0, z1, fz = _axis_coords(cz, z_dim - 1)
                xb = (x0 * (y_dim * z_dim) + boff, x1 * (y_dim * z_dim) + boff)
                yb = (y0 * z_dim, y1 * z_dim)
                zb = (z0, z1)
                fr_r[0, pl.ds(g * L, L)] = fx
                fr_r[1, pl.ds(g * L, L)] = fy
                fr_r[2, pl.ds(g * L, L)] = fz
                for dx in (0, 1):
                    xyb = (xb[dx] + yb[0], xb[dx] + yb[1])
                    for dy in (0, 1):
                        for dz in (0, 1):
                            k = dx * 4 + dy * 2 + dz
                            idx_r[k, pl.ds(g * L, L)] = xyb[dy] + zb[dz]

        def fire(c, idx_r, rows_r, cv_r, sem):
            for k in range(8):
                pltpu.async_copy(gflat.at[idx_r.at[k]], rows_r.at[k], sem)
            fetch_coords(c + 2, cv_r)

        def drain(c, idx_r, rows_r, cv_r, sem):
            for k in range(8):
                pltpu.make_async_copy(gflat.at[idx_r.at[k]], rows_r.at[k],
                                      sem).wait()
            o = sb + coff(c + 2)
            pltpu.make_async_copy(xs.at[batch, pl.ds(o, CHUNK)],
                                  cv_r.at[0], sem).wait()
            pltpu.make_async_copy(ys.at[batch, pl.ds(o, CHUNK)],
                                  cv_r.at[1], sem).wait()
            pltpu.make_async_copy(zs.at[batch, pl.ds(o, CHUNK)],
                                  cv_r.at[2], sem).wait()

        def accum(c, rows_r, fr_r, ob):
            def pbody(p, carry):
                gb = (p // L) * L
                lane = p - gb
                bx = _bcast_lane(fr_r[0, pl.ds(gb, L)], lane)
                by = _bcast_lane(fr_r[1, pl.ds(gb, L)], lane)
                bz = _bcast_lane(fr_r[2, pl.ds(gb, L)], lane)
                wx = (1.0 - bx, bx)
                wyz = ((1.0 - by) * (1.0 - bz), (1.0 - by) * bz,
                       by * (1.0 - bz), by * bz)
                w0 = wx[0] * wyz[0]
                a0 = rows_r[0, p, pl.ds(0, L)] * w0
                a1 = rows_r[0, p, pl.ds(L, L)] * w0
                for k in range(1, 8):
                    wk = wx[k // 4] * wyz[k % 4]
                    a0 = a0 + rows_r[k, p, pl.ds(0, L)] * wk
                    a1 = a1 + rows_r[k, p, pl.ds(L, L)] * wk
                ob[p, pl.ds(0, L)] = a0
                ob[p, pl.ds(L, L)] = a1
                return carry

            lax.fori_loop(0, CHUNK, pbody, 0, unroll=2)
            full = c * CHUNK + CHUNK <= slen

            @pl.when(full)
            def _full():
                pltpu.sync_copy(
                    ob, out.at[batch, pl.ds(orow0 + c * CHUNK, CHUNK)])

            if part1:
                @pl.when(jnp.logical_not(full))
                def _partial():
                    pltpu.sync_copy(
                        ob.at[pl.ds(CHUNK - part1, part1)],
                        out.at[batch, pl.ds(orow0 + slen - part1, part1)])

        def pair(i, carry):
            c0 = i * 2
            stage(idxB, cvB, frB)
            fire(c0 + 1, idxB, rowsB, cvB, semB)
            drain(c0, idxA, rowsA, cvA, semA)
            accum(c0, rowsA, frA, outA)

            @pl.when(i + 1 < npairs)
            def _next_even():
                stage(idxA, cvA, frA)
                fire(c0 + 2, idxA, rowsA, cvA, semA)

            drain(c0 + 1, idxB, rowsB, cvB, semB)
            accum(c0 + 1, rowsB, frB, outB)
            return carry

        fetch_coords_sync(0, cvA)
        fetch_coords_sync(1, cvB)
        stage(idxA, cvA, frA)
        fire(0, idxA, rowsA, cvA, semA)
        lax.fori_loop(0, npairs, pair, 0)

    return sc_call


def kernel(grid, locs):
    B, X, Y, Z, C = grid.shape
    N = locs.shape[1]
    gmid = jax.lax.optimization_barrier(grid.reshape(B, X, Y, Z * C))
    gflat = gmid.reshape(B * X * Y * Z, C)
    locs32 = locs.astype(jnp.float32)
    xs = locs32[:, :, 0]
    ys = locs32[:, :, 1]
    zs = locs32[:, :, 2]
    sc_call = _make_sc_call(B, N, X, Y, Z, C)
    return sc_call(gflat, xs, ys, zs)
